# baseline (speedup 1.0000x reference)
import functools

import jax
import jax.numpy as jnp
from jax import lax
from jax.experimental import pallas as pl
from jax.experimental.pallas import tpu as pltpu
from jax.experimental.pallas import tpu_sc as plsc

N = 10000
E = 320000
D_IN = 128
D_H = 256
D_OUT = 40
BN_EPS = 1e-5

NPAD = 10240
NW = 32
K = 80
EPW = E // NW
NBUF = 2
CHUNKS = EPW // K
ROWS_PER_TILE = NPAD // 16
R = 256


def _hist_body(dst_e, zeros1, out, ibuf, ones_v, acc, isem):
    c = lax.axis_index("c")
    s = lax.axis_index("s")
    w = s * 2 + c
    base = w * EPW
    lo = s * ROWS_PER_TILE
    for i in range(K // 16):
        ones_v[pl.ds(i * 16, 16)] = jnp.ones((16,), jnp.float32)
    pltpu.sync_copy(zeros1.at[pl.ds(lo, ROWS_PER_TILE)],
                    acc.at[pl.ds(lo, ROWS_PER_TILE)])
    plsc.subcore_barrier()

    for p in range(4):
        pltpu.async_copy(dst_e.at[pl.ds(base + p * K, K)], ibuf.at[p], isem)

    @pl.loop(0, CHUNKS)
    def _(j):
        ib = lax.rem(j, 4)
        pltpu.make_async_copy(dst_e.at[pl.ds(0, K)], ibuf.at[ib],
                              isem).wait()
        pltpu.sync_copy(ones_v, acc.at[ibuf.at[ib]], add=True)

        @pl.when(j + 4 < CHUNKS)
        def _():
            pltpu.async_copy(dst_e.at[pl.ds(base + (j + 4) * K, K)],
                             ibuf.at[ib], isem)

    plsc.subcore_barrier()
    pltpu.sync_copy(acc.at[pl.ds(lo, ROWS_PER_TILE)],
                    out.at[c, pl.ds(lo, ROWS_PER_TILE)])


_hist = functools.partial(
    pl.kernel,
    out_type=jax.ShapeDtypeStruct((2, NPAD), jnp.float32),
    mesh=plsc.VectorSubcoreMesh(core_axis_name="c", subcore_axis_name="s"),
    scratch_types=[
        pltpu.VMEM((4, K), jnp.int32),
        pltpu.VMEM((K,), jnp.float32),
        pltpu.VMEM_SHARED((NPAD,), jnp.float32),
        pltpu.SemaphoreType.DMA,
    ],
)(_hist_body)


def _pair(src_e, dst_e, base, j, ibuf, ib, isem):
    pltpu.async_copy(src_e.at[pl.ds(base + j * K, K)], ibuf.at[ib, 0], isem)
    pltpu.async_copy(dst_e.at[pl.ds(base + j * K, K)], ibuf.at[ib, 1], isem)


def _pair_wait(src_e, ibuf, ib, isem):
    pltpu.make_async_copy(src_e.at[pl.ds(0, K)], ibuf.at[ib, 0], isem).wait()
    pltpu.make_async_copy(src_e.at[pl.ds(0, K)], ibuf.at[ib, 1], isem).wait()


def _one_pass(table, zeros, src_e, dst_e, out, ibuf, rows, acc, gsem, isem,
              c, w, lo):
    base = w * EPW

    @pl.when(c == 0)
    def _():
        pltpu.sync_copy(table.at[pl.ds(lo, ROWS_PER_TILE)],
                        acc.at[pl.ds(lo, ROWS_PER_TILE)])

    @pl.when(c != 0)
    def _():
        pltpu.sync_copy(zeros.at[pl.ds(lo, ROWS_PER_TILE)],
                        acc.at[pl.ds(lo, ROWS_PER_TILE)])

    plsc.subcore_barrier()

    for p in range(4):
        _pair(src_e, dst_e, base, p, ibuf, p, isem)
    for p in range(NBUF):
        _pair_wait(src_e, ibuf, p, isem)
        pltpu.async_copy(table.at[ibuf.at[p, 0]], rows.at[p], gsem)

    def chunk(j, b, ib):
        pltpu.make_async_copy(table.at[pl.ds(0, K)], rows.at[b],
                              gsem).wait()
        pltpu.sync_copy(rows.at[b], acc.at[ibuf.at[ib, 1]], add=True)

        @pl.when(j + 4 < CHUNKS)
        def _():
            _pair(src_e, dst_e, base, j + 4, ibuf, ib, isem)

        @pl.when(j + NBUF < CHUNKS)
        def _():
            ib2 = lax.rem(j + NBUF, 4)
            _pair_wait(src_e, ibuf, ib2, isem)
            pltpu.async_copy(table.at[ibuf.at[ib2, 0]], rows.at[b], gsem)

    @pl.loop(0, CHUNKS - 1, step=NBUF)
    def _(i):
        for b in range(NBUF):
            chunk(i + b, b, lax.rem(i + b, 4))

    chunk(CHUNKS - 1, (CHUNKS - 1) % NBUF, (CHUNKS - 1) % 4)

    plsc.subcore_barrier()
    pltpu.sync_copy(acc.at[pl.ds(lo, ROWS_PER_TILE)],
                    out.at[c, pl.ds(lo, ROWS_PER_TILE)])


def _make_scatter(D, n_tables=1, tc_tiling=True):
    def body(*args):
        tables = args[:n_tables]
        zeros, src_e, dst_e = args[n_tables:n_tables + 3]
        outs = args[n_tables + 3:n_tables + 3 + n_tables]
        ibuf, rows, acc, gsem, isem = args[n_tables + 3 + n_tables:]
        c = lax.axis_index("c")
        s = lax.axis_index("s")
        w = s * 2 + c
        lo = s * ROWS_PER_TILE
        for t in range(n_tables):
            _one_pass(tables[t], zeros, src_e, dst_e, outs[t], ibuf, rows,
                      acc, gsem, isem, c, w, lo)

    out_t = jax.ShapeDtypeStruct((2, NPAD, D), jnp.float32)
    return functools.partial(
        pl.kernel,
        out_type=out_t if n_tables == 1 else [out_t] * n_tables,
        mesh=plsc.VectorSubcoreMesh(core_axis_name="c", subcore_axis_name="s"),
        compiler_params=None if tc_tiling else pltpu.CompilerParams(
            use_tc_tiling_on_sc=False),
        scratch_types=[
            pltpu.VMEM((4, 2, K), jnp.int32),
            pltpu.VMEM((NBUF, K, D), jnp.float32),
            pltpu.VMEM_SHARED((NPAD, D), jnp.float32),
            pltpu.SemaphoreType.DMA,
            pltpu.SemaphoreType.DMA,
        ],
    )(body)


_scatter128 = _make_scatter(128)
_scatter128x2 = _make_scatter(128, n_tables=2)
_scatter64 = _make_scatter(64, tc_tiling=False)


def _prep_body(deg2_ref, x_ref, dinv_ref, g0_ref):
    deg = deg2_ref[0] + deg2_ref[1] + 1.0
    dv = lax.rsqrt(deg)
    dinv_ref[...] = dv
    g0_ref[...] = x_ref[...] * dv


def _prep(deg2, x_pad):
    return pl.pallas_call(
        _prep_body,
        grid=(NPAD // R,),
        in_specs=[
            pl.BlockSpec((2, R, 1), lambda i: (0, i, 0)),
            pl.BlockSpec((R, D_IN), lambda i: (i, 0)),
        ],
        out_specs=[
            pl.BlockSpec((R, 1), lambda i: (i, 0)),
            pl.BlockSpec((R, D_IN), lambda i: (i, 0)),
        ],
        out_shape=[
            jax.ShapeDtypeStruct((NPAD, 1), jnp.float32),
            jax.ShapeDtypeStruct((NPAD, D_IN), jnp.float32),
        ],
    )(deg2, x_pad)


def _layer1_body(s0_ref, dinv_ref, w1_ref, sc1_ref, sh1_ref, ga_ref, gb_ref):
    dv = dinv_ref[...]
    a1 = (s0_ref[0] + s0_ref[1]) * dv
    h = jnp.dot(a1, w1_ref[...], preferred_element_type=jnp.float32)
    h = jnp.maximum(h * sc1_ref[...] + sh1_ref[...], 0.0)
    g1 = h * dv
    ga_ref[...] = g1[:, :128]
    gb_ref[...] = g1[:, 128:]


def _layer1(s0, dinv, w1, sc1, sh1):
    return pl.pallas_call(
        _layer1_body,
        grid=(NPAD // R,),
        in_specs=[
            pl.BlockSpec((2, R, 128), lambda i: (0, i, 0)),
            pl.BlockSpec((R, 1), lambda i: (i, 0)),
            pl.BlockSpec((D_IN, D_H), lambda i: (0, 0)),
            pl.BlockSpec((1, D_H), lambda i: (0, 0)),
            pl.BlockSpec((1, D_H), lambda i: (0, 0)),
        ],
        out_specs=[
            pl.BlockSpec((R, 128), lambda i: (i, 0)),
            pl.BlockSpec((R, 128), lambda i: (i, 0)),
        ],
        out_shape=[
            jax.ShapeDtypeStruct((NPAD, 128), jnp.float32),
            jax.ShapeDtypeStruct((NPAD, 128), jnp.float32),
        ],
    )(s0, dinv, w1, sc1, sh1)


def _layer23_body(s1a_ref, s1b_ref, dinv_ref, w2_ref, w3_ref, sc2_ref,
                  sh2_ref, g2_ref):
    dv = dinv_ref[...]
    a2 = jnp.concatenate([s1a_ref[0] + s1a_ref[1],
                          s1b_ref[0] + s1b_ref[1]], axis=1) * dv
    h = jnp.dot(a2, w2_ref[...], preferred_element_type=jnp.float32)
    h = jnp.maximum(h * sc2_ref[...] + sh2_ref[...], 0.0)
    z = jnp.dot(h, w3_ref[...], preferred_element_type=jnp.float32)
    g2_ref[...] = z * dv


def _layer23(s1a, s1b, dinv, w2, w3p, sc2, sh2):
    return pl.pallas_call(
        _layer23_body,
        grid=(NPAD // R,),
        in_specs=[
            pl.BlockSpec((2, R, 128), lambda i: (0, i, 0)),
            pl.BlockSpec((2, R, 128), lambda i: (0, i, 0)),
            pl.BlockSpec((R, 1), lambda i: (i, 0)),
            pl.BlockSpec((D_H, D_H), lambda i: (0, 0)),
            pl.BlockSpec((D_H, 64), lambda i: (0, 0)),
            pl.BlockSpec((1, D_H), lambda i: (0, 0)),
            pl.BlockSpec((1, D_H), lambda i: (0, 0)),
        ],
        out_specs=pl.BlockSpec((R, 64), lambda i: (i, 0)),
        out_shape=jax.ShapeDtypeStruct((NPAD, 64), jnp.float32),
    )(s1a, s1b, dinv, w2, w3p, sc2, sh2)


def _final_body(s2_ref, dinv_ref, b3_ref, out_ref):
    a3 = (s2_ref[0] + s2_ref[1]) * dinv_ref[...] + b3_ref[...]
    m = jnp.max(a3, axis=1, keepdims=True)
    e = jnp.exp(a3 - m)
    lse = jnp.log(jnp.sum(e, axis=1, keepdims=True))
    out_ref[...] = (a3 - m - lse)[:, :D_OUT]


def _final(s2, dinv, b3p):
    return pl.pallas_call(
        _final_body,
        grid=(NPAD // R,),
        in_specs=[
            pl.BlockSpec((2, R, 64), lambda i: (0, i, 0)),
            pl.BlockSpec((R, 1), lambda i: (i, 0)),
            pl.BlockSpec((1, 64), lambda i: (0, 0)),
        ],
        out_specs=pl.BlockSpec((R, D_OUT), lambda i: (i, 0)),
        out_shape=jax.ShapeDtypeStruct((N, D_OUT), jnp.float32),
    )(s2, dinv, b3p)


def kernel(x, edge_index, W1, b1, W2, b2, W3, b3,
           bn1_w, bn1_b, bn2_w, bn2_b):
    f32 = jnp.float32
    x_pad = jnp.pad(x, ((0, NPAD - N), (0, 0)))

    c1 = bn1_w * (1.0 / jnp.sqrt(1.0 + BN_EPS))
    sc1 = c1.reshape(1, D_H)
    sh1 = (b1 * c1 + bn1_b).reshape(1, D_H)
    c2 = bn2_w * (1.0 / jnp.sqrt(1.0 + BN_EPS))
    sc2 = c2.reshape(1, D_H)
    sh2 = (b2 * c2 + bn2_b).reshape(1, D_H)
    w3p = jnp.pad(W3, ((0, 0), (0, 64 - D_OUT)))
    b3p = jnp.concatenate([b3, jnp.full((64 - D_OUT,), -1e30, f32)]
                          ).reshape(1, 64)

    zeros1 = jnp.zeros((NPAD,), f32)
    zeros128 = jnp.zeros((NPAD, 128), f32)
    zeros64 = jnp.zeros((NPAD, 64), f32)

    src_e = edge_index[0]
    dst_e = edge_index[1]
    deg2 = _hist(dst_e, zeros1)
    dinv, g0 = _prep(deg2[:, :, None], x_pad)
    s0 = _scatter128(g0, zeros128, src_e, dst_e)
    g1a, g1b = _layer1(s0, dinv, W1, sc1, sh1)
    s1a, s1b = _scatter128x2(g1a, g1b, zeros128, src_e, dst_e)
    g2 = _layer23(s1a, s1b, dinv, W2, w3p, sc2, sh2)
    s2 = _scatter64(g2, zeros64, src_e, dst_e)
    return _final(s2, dinv, b3p)


# speedup vs baseline: 1.0788x; 1.0788x over previous
import functools

import jax
import jax.numpy as jnp
from jax import lax
from jax.experimental import pallas as pl
from jax.experimental.pallas import tpu as pltpu
from jax.experimental.pallas import tpu_sc as plsc

N = 10000
E = 320000
D_IN = 128
D_H = 256
D_OUT = 40
BN_EPS = 1e-5

NPAD = 10240
NW = 32
K = 128
EPW = E // NW
NBUF = 2
CHUNKS = 80
EPW_PAD = CHUNKS * K
ROWS_PER_TILE = NPAD // 16
R = 256


def _hist_body(idxs, zeros1, out, ibuf, ones_v, acc, isem):
    c = lax.axis_index("c")
    s = lax.axis_index("s")
    w = s * 2 + c
    lo = s * ROWS_PER_TILE
    for i in range(K // 16):
        ones_v[pl.ds(i * 16, 16)] = jnp.ones((16,), jnp.float32)
    pltpu.sync_copy(zeros1.at[pl.ds(lo, ROWS_PER_TILE)],
                    acc.at[pl.ds(lo, ROWS_PER_TILE)])
    plsc.subcore_barrier()

    for p in range(4):
        pltpu.async_copy(idxs.at[w, p], ibuf.at[p], isem)

    @pl.loop(0, CHUNKS)
    def _(j):
        ib = lax.rem(j, 4)
        pltpu.make_async_copy(idxs.at[w, 0], ibuf.at[ib], isem).wait()
        pltpu.sync_copy(ones_v, acc.at[ibuf.at[ib, 1]], add=True)

        @pl.when(j + 4 < CHUNKS)
        def _():
            pltpu.async_copy(idxs.at[w, j + 4], ibuf.at[ib], isem)

    plsc.subcore_barrier()
    pltpu.sync_copy(acc.at[pl.ds(lo, ROWS_PER_TILE)],
                    out.at[c, pl.ds(lo, ROWS_PER_TILE)])


_hist = functools.partial(
    pl.kernel,
    out_type=jax.ShapeDtypeStruct((2, NPAD), jnp.float32),
    mesh=plsc.VectorSubcoreMesh(core_axis_name="c", subcore_axis_name="s"),
    scratch_types=[
        pltpu.VMEM((4, 2, K), jnp.int32),
        pltpu.VMEM((K,), jnp.float32),
        pltpu.VMEM_SHARED((NPAD,), jnp.float32),
        pltpu.SemaphoreType.DMA,
    ],
)(_hist_body)


def _one_pass(table, zeros, idxs, out, ibuf, rows, acc, gsem, isem, c, w, lo):
    @pl.when(c == 0)
    def _():
        pltpu.sync_copy(table.at[pl.ds(lo, ROWS_PER_TILE)],
                        acc.at[pl.ds(lo, ROWS_PER_TILE)])

    @pl.when(c != 0)
    def _():
        pltpu.sync_copy(zeros.at[pl.ds(lo, ROWS_PER_TILE)],
                        acc.at[pl.ds(lo, ROWS_PER_TILE)])

    plsc.subcore_barrier()

    for p in range(4):
        pltpu.async_copy(idxs.at[w, p], ibuf.at[p], isem)
    for p in range(NBUF):
        pltpu.make_async_copy(idxs.at[w, 0], ibuf.at[p], isem).wait()
        pltpu.async_copy(table.at[ibuf.at[p, 0]], rows.at[p], gsem)

    @pl.loop(0, CHUNKS, step=NBUF)
    def _(i):
        for b in range(NBUF):
            j = i + b
            ib = lax.rem(j, 4)
            pltpu.make_async_copy(table.at[pl.ds(0, K)], rows.at[b],
                                  gsem).wait()
            pltpu.sync_copy(rows.at[b], acc.at[ibuf.at[ib, 1]], add=True)

            @pl.when(j + 4 < CHUNKS)
            def _():
                pltpu.async_copy(idxs.at[w, j + 4], ibuf.at[ib], isem)

            @pl.when(j + NBUF < CHUNKS)
            def _():
                ib2 = lax.rem(j + NBUF, 4)
                pltpu.make_async_copy(idxs.at[w, 0], ibuf.at[ib2],
                                      isem).wait()
                pltpu.async_copy(table.at[ibuf.at[ib2, 0]], rows.at[b],
                                 gsem)

    plsc.subcore_barrier()
    pltpu.sync_copy(acc.at[pl.ds(lo, ROWS_PER_TILE)],
                    out.at[c, pl.ds(lo, ROWS_PER_TILE)])


def _make_scatter(D, n_tables=1, tc_tiling=True):
    def body(*args):
        tables = args[:n_tables]
        zeros, idxs = args[n_tables:n_tables + 2]
        outs = args[n_tables + 2:n_tables + 2 + n_tables]
        ibuf, rows, acc, gsem, isem = args[n_tables + 2 + n_tables:]
        c = lax.axis_index("c")
        s = lax.axis_index("s")
        w = s * 2 + c
        lo = s * ROWS_PER_TILE
        for t in range(n_tables):
            _one_pass(tables[t], zeros, idxs, outs[t], ibuf, rows, acc,
                      gsem, isem, c, w, lo)

    out_t = jax.ShapeDtypeStruct((2, NPAD, D), jnp.float32)
    return functools.partial(
        pl.kernel,
        out_type=out_t if n_tables == 1 else [out_t] * n_tables,
        mesh=plsc.VectorSubcoreMesh(core_axis_name="c", subcore_axis_name="s"),
        compiler_params=None if tc_tiling else pltpu.CompilerParams(
            use_tc_tiling_on_sc=False),
        scratch_types=[
            pltpu.VMEM((4, 2, K), jnp.int32),
            pltpu.VMEM((NBUF, K, D), jnp.float32),
            pltpu.VMEM_SHARED((NPAD, D), jnp.float32),
            pltpu.SemaphoreType.DMA,
            pltpu.SemaphoreType.DMA,
        ],
    )(body)


_scatter128 = _make_scatter(128)
_scatter128x2 = _make_scatter(128, n_tables=2)
_scatter64 = _make_scatter(64, tc_tiling=False)


def _prep_body(deg2_ref, x_ref, dinv_ref, g0_ref):
    deg = deg2_ref[0] + deg2_ref[1] + 1.0
    dv = lax.rsqrt(deg)
    dinv_ref[...] = dv
    g0_ref[...] = x_ref[...] * dv


def _prep(deg2, x_pad):
    return pl.pallas_call(
        _prep_body,
        grid=(NPAD // R,),
        in_specs=[
            pl.BlockSpec((2, R, 1), lambda i: (0, i, 0)),
            pl.BlockSpec((R, D_IN), lambda i: (i, 0)),
        ],
        out_specs=[
            pl.BlockSpec((R, 1), lambda i: (i, 0)),
            pl.BlockSpec((R, D_IN), lambda i: (i, 0)),
        ],
        out_shape=[
            jax.ShapeDtypeStruct((NPAD, 1), jnp.float32),
            jax.ShapeDtypeStruct((NPAD, D_IN), jnp.float32),
        ],
    )(deg2, x_pad)


def _layer1_body(s0_ref, dinv_ref, w1_ref, sc1_ref, sh1_ref, ga_ref, gb_ref):
    dv = dinv_ref[...]
    a1 = (s0_ref[0] + s0_ref[1]) * dv
    h = jnp.dot(a1, w1_ref[...], preferred_element_type=jnp.float32)
    h = jnp.maximum(h * sc1_ref[...] + sh1_ref[...], 0.0)
    g1 = h * dv
    ga_ref[...] = g1[:, :128]
    gb_ref[...] = g1[:, 128:]


def _layer1(s0, dinv, w1, sc1, sh1):
    return pl.pallas_call(
        _layer1_body,
        grid=(NPAD // R,),
        in_specs=[
            pl.BlockSpec((2, R, 128), lambda i: (0, i, 0)),
            pl.BlockSpec((R, 1), lambda i: (i, 0)),
            pl.BlockSpec((D_IN, D_H), lambda i: (0, 0)),
            pl.BlockSpec((1, D_H), lambda i: (0, 0)),
            pl.BlockSpec((1, D_H), lambda i: (0, 0)),
        ],
        out_specs=[
            pl.BlockSpec((R, 128), lambda i: (i, 0)),
            pl.BlockSpec((R, 128), lambda i: (i, 0)),
        ],
        out_shape=[
            jax.ShapeDtypeStruct((NPAD, 128), jnp.float32),
            jax.ShapeDtypeStruct((NPAD, 128), jnp.float32),
        ],
    )(s0, dinv, w1, sc1, sh1)


def _layer23_body(s1a_ref, s1b_ref, dinv_ref, w2_ref, w3_ref, sc2_ref,
                  sh2_ref, g2_ref):
    dv = dinv_ref[...]
    a2 = jnp.concatenate([s1a_ref[0] + s1a_ref[1],
                          s1b_ref[0] + s1b_ref[1]], axis=1) * dv
    h = jnp.dot(a2, w2_ref[...], preferred_element_type=jnp.float32)
    h = jnp.maximum(h * sc2_ref[...] + sh2_ref[...], 0.0)
    z = jnp.dot(h, w3_ref[...], preferred_element_type=jnp.float32)
    g2_ref[...] = z * dv


def _layer23(s1a, s1b, dinv, w2, w3p, sc2, sh2):
    return pl.pallas_call(
        _layer23_body,
        grid=(NPAD // R,),
        in_specs=[
            pl.BlockSpec((2, R, 128), lambda i: (0, i, 0)),
            pl.BlockSpec((2, R, 128), lambda i: (0, i, 0)),
            pl.BlockSpec((R, 1), lambda i: (i, 0)),
            pl.BlockSpec((D_H, D_H), lambda i: (0, 0)),
            pl.BlockSpec((D_H, 64), lambda i: (0, 0)),
            pl.BlockSpec((1, D_H), lambda i: (0, 0)),
            pl.BlockSpec((1, D_H), lambda i: (0, 0)),
        ],
        out_specs=pl.BlockSpec((R, 64), lambda i: (i, 0)),
        out_shape=jax.ShapeDtypeStruct((NPAD, 64), jnp.float32),
    )(s1a, s1b, dinv, w2, w3p, sc2, sh2)


def _final_body(s2_ref, dinv_ref, b3_ref, out_ref):
    a3 = (s2_ref[0] + s2_ref[1]) * dinv_ref[...] + b3_ref[...]
    m = jnp.max(a3, axis=1, keepdims=True)
    e = jnp.exp(a3 - m)
    lse = jnp.log(jnp.sum(e, axis=1, keepdims=True))
    out_ref[...] = (a3 - m - lse)[:, :D_OUT]


def _final(s2, dinv, b3p):
    return pl.pallas_call(
        _final_body,
        grid=(NPAD // R,),
        in_specs=[
            pl.BlockSpec((2, R, 64), lambda i: (0, i, 0)),
            pl.BlockSpec((R, 1), lambda i: (i, 0)),
            pl.BlockSpec((1, 64), lambda i: (0, 0)),
        ],
        out_specs=pl.BlockSpec((R, D_OUT), lambda i: (i, 0)),
        out_shape=jax.ShapeDtypeStruct((N, D_OUT), jnp.float32),
    )(s2, dinv, b3p)


def kernel(x, edge_index, W1, b1, W2, b2, W3, b3,
           bn1_w, bn1_b, bn2_w, bn2_b):
    f32 = jnp.float32
    x_pad = jnp.pad(x, ((0, NPAD - N), (0, 0)))

    src = edge_index[0].reshape(NW, EPW)
    dst = edge_index[1].reshape(NW, EPW)
    n_fill = EPW_PAD - EPW
    fill = (N + (jnp.arange(NW * n_fill, dtype=jnp.int32) % (NPAD - N))
            ).reshape(NW, n_fill)
    srcs = jnp.concatenate([src, fill], axis=1).reshape(NW, CHUNKS, K)
    dsts = jnp.concatenate([dst, fill], axis=1).reshape(NW, CHUNKS, K)
    idxs = jnp.stack([srcs, dsts], axis=2)

    c1 = bn1_w * (1.0 / jnp.sqrt(1.0 + BN_EPS))
    sc1 = c1.reshape(1, D_H)
    sh1 = (b1 * c1 + bn1_b).reshape(1, D_H)
    c2 = bn2_w * (1.0 / jnp.sqrt(1.0 + BN_EPS))
    sc2 = c2.reshape(1, D_H)
    sh2 = (b2 * c2 + bn2_b).reshape(1, D_H)
    w3p = jnp.pad(W3, ((0, 0), (0, 64 - D_OUT)))
    b3p = jnp.concatenate([b3, jnp.full((64 - D_OUT,), -1e30, f32)]
                          ).reshape(1, 64)

    zeros1 = jnp.zeros((NPAD,), f32)
    zeros128 = jnp.zeros((NPAD, 128), f32)
    zeros64 = jnp.zeros((NPAD, 64), f32)

    deg2 = _hist(idxs, zeros1)
    dinv, g0 = _prep(deg2[:, :, None], x_pad)
    s0 = _scatter128(g0, zeros128, idxs)
    g1a, g1b = _layer1(s0, dinv, W1, sc1, sh1)
    s1a, s1b = _scatter128x2(g1a, g1b, zeros128, idxs)
    g2 = _layer23(s1a, s1b, dinv, W2, w3p, sc2, sh2)
    s2 = _scatter64(g2, zeros64, idxs)
    return _final(s2, dinv, b3p)


# speedup vs baseline: 1.1629x; 1.0780x over previous
import functools

import jax
import jax.numpy as jnp
from jax import lax
from jax.experimental import pallas as pl
from jax.experimental.pallas import tpu as pltpu
from jax.experimental.pallas import tpu_sc as plsc

N = 10000
E = 320000
D_IN = 128
D_H = 256
D_OUT = 40
BN_EPS = 1e-5

NPAD = 10240
NW = 32
K = 128
EPW = E // NW
NBUF = 2
CHUNKS = 80
EPW_PAD = CHUNKS * K
ROWS_PER_TILE = NPAD // 16
R = 512


def _hist_body(idxs, zeros1, out, ibuf, ones_v, acc, isem):
    c = lax.axis_index("c")
    s = lax.axis_index("s")
    w = s * 2 + c
    lo = s * ROWS_PER_TILE
    for i in range(K // 16):
        ones_v[pl.ds(i * 16, 16)] = jnp.ones((16,), jnp.float32)
    pltpu.sync_copy(zeros1.at[pl.ds(lo, ROWS_PER_TILE)],
                    acc.at[pl.ds(lo, ROWS_PER_TILE)])
    plsc.subcore_barrier()

    for p in range(4):
        pltpu.async_copy(idxs.at[w, p], ibuf.at[p], isem)

    @pl.loop(0, CHUNKS)
    def _(j):
        ib = lax.rem(j, 4)
        pltpu.make_async_copy(idxs.at[w, 0], ibuf.at[ib], isem).wait()
        pltpu.sync_copy(ones_v, acc.at[ibuf.at[ib, 1]], add=True)

        @pl.when(j + 4 < CHUNKS)
        def _():
            pltpu.async_copy(idxs.at[w, j + 4], ibuf.at[ib], isem)

    plsc.subcore_barrier()
    pltpu.sync_copy(acc.at[pl.ds(lo, ROWS_PER_TILE)],
                    out.at[c, pl.ds(lo, ROWS_PER_TILE)])


_hist = functools.partial(
    pl.kernel,
    out_type=jax.ShapeDtypeStruct((2, NPAD), jnp.float32),
    mesh=plsc.VectorSubcoreMesh(core_axis_name="c", subcore_axis_name="s"),
    scratch_types=[
        pltpu.VMEM((4, 2, K), jnp.int32),
        pltpu.VMEM((K,), jnp.float32),
        pltpu.VMEM_SHARED((NPAD,), jnp.float32),
        pltpu.SemaphoreType.DMA,
    ],
)(_hist_body)


def _one_pass(table, zeros, idxs, out, ibuf, rows, acc, gsem, isem, c, w, lo):
    @pl.when(c == 0)
    def _():
        pltpu.sync_copy(table.at[pl.ds(lo, ROWS_PER_TILE)],
                        acc.at[pl.ds(lo, ROWS_PER_TILE)])

    @pl.when(c != 0)
    def _():
        pltpu.sync_copy(zeros.at[pl.ds(lo, ROWS_PER_TILE)],
                        acc.at[pl.ds(lo, ROWS_PER_TILE)])

    plsc.subcore_barrier()

    for p in range(4):
        pltpu.async_copy(idxs.at[w, p], ibuf.at[p], isem)
    for p in range(NBUF):
        pltpu.make_async_copy(idxs.at[w, 0], ibuf.at[p], isem).wait()
        pltpu.async_copy(table.at[ibuf.at[p, 0]], rows.at[p], gsem)

    @pl.loop(0, CHUNKS, step=NBUF)
    def _(i):
        for b in range(NBUF):
            j = i + b
            ib = lax.rem(j, 4)
            pltpu.make_async_copy(table.at[pl.ds(0, K)], rows.at[b],
                                  gsem).wait()
            pltpu.sync_copy(rows.at[b], acc.at[ibuf.at[ib, 1]], add=True)

            @pl.when(j + 4 < CHUNKS)
            def _():
                pltpu.async_copy(idxs.at[w, j + 4], ibuf.at[ib], isem)

            @pl.when(j + NBUF < CHUNKS)
            def _():
                ib2 = lax.rem(j + NBUF, 4)
                pltpu.make_async_copy(idxs.at[w, 0], ibuf.at[ib2],
                                      isem).wait()
                pltpu.async_copy(table.at[ibuf.at[ib2, 0]], rows.at[b],
                                 gsem)

    plsc.subcore_barrier()
    pltpu.sync_copy(acc.at[pl.ds(lo, ROWS_PER_TILE)],
                    out.at[c, pl.ds(lo, ROWS_PER_TILE)])


def _make_scatter(D, n_tables=1, tc_tiling=True):
    def body(*args):
        tables = args[:n_tables]
        zeros, idxs = args[n_tables:n_tables + 2]
        outs = args[n_tables + 2:n_tables + 2 + n_tables]
        ibuf, rows, acc, gsem, isem = args[n_tables + 2 + n_tables:]
        c = lax.axis_index("c")
        s = lax.axis_index("s")
        w = s * 2 + c
        lo = s * ROWS_PER_TILE
        for t in range(n_tables):
            _one_pass(tables[t], zeros, idxs, outs[t], ibuf, rows, acc,
                      gsem, isem, c, w, lo)

    out_t = jax.ShapeDtypeStruct((2, NPAD, D), jnp.float32)
    return functools.partial(
        pl.kernel,
        out_type=out_t if n_tables == 1 else [out_t] * n_tables,
        mesh=plsc.VectorSubcoreMesh(core_axis_name="c", subcore_axis_name="s"),
        compiler_params=None if tc_tiling else pltpu.CompilerParams(
            use_tc_tiling_on_sc=False),
        scratch_types=[
            pltpu.VMEM((4, 2, K), jnp.int32),
            pltpu.VMEM((NBUF, K, D), jnp.float32),
            pltpu.VMEM_SHARED((NPAD, D), jnp.float32),
            pltpu.SemaphoreType.DMA,
            pltpu.SemaphoreType.DMA,
        ],
    )(body)


_scatter128 = _make_scatter(128)
_scatter128x2 = _make_scatter(128, n_tables=2)
_scatter64 = _make_scatter(64, tc_tiling=False)


def _prep_body(deg2_ref, x_ref, dinv_ref, g0_ref):
    deg = deg2_ref[0] + deg2_ref[1] + 1.0
    dv = lax.rsqrt(deg)
    dinv_ref[...] = dv
    g0_ref[...] = x_ref[...] * dv


def _prep(deg2, x_pad):
    return pl.pallas_call(
        _prep_body,
        grid=(NPAD // R,),
        in_specs=[
            pl.BlockSpec((2, R, 1), lambda i: (0, i, 0)),
            pl.BlockSpec((R, D_IN), lambda i: (i, 0)),
        ],
        out_specs=[
            pl.BlockSpec((R, 1), lambda i: (i, 0)),
            pl.BlockSpec((R, D_IN), lambda i: (i, 0)),
        ],
        out_shape=[
            jax.ShapeDtypeStruct((NPAD, 1), jnp.float32),
            jax.ShapeDtypeStruct((NPAD, D_IN), jnp.float32),
        ],
    )(deg2, x_pad)


def _layer1_body(s0_ref, dinv_ref, w1_ref, sc1_ref, sh1_ref, ga_ref, gb_ref):
    dv = dinv_ref[...]
    a1 = (s0_ref[0] + s0_ref[1]) * dv
    h = jnp.dot(a1, w1_ref[...], preferred_element_type=jnp.float32)
    h = jnp.maximum(h * sc1_ref[...] + sh1_ref[...], 0.0)
    g1 = h * dv
    ga_ref[...] = g1[:, :128]
    gb_ref[...] = g1[:, 128:]


def _layer1(s0, dinv, w1, sc1, sh1):
    return pl.pallas_call(
        _layer1_body,
        grid=(NPAD // R,),
        in_specs=[
            pl.BlockSpec((2, R, 128), lambda i: (0, i, 0)),
            pl.BlockSpec((R, 1), lambda i: (i, 0)),
            pl.BlockSpec((D_IN, D_H), lambda i: (0, 0)),
            pl.BlockSpec((1, D_H), lambda i: (0, 0)),
            pl.BlockSpec((1, D_H), lambda i: (0, 0)),
        ],
        out_specs=[
            pl.BlockSpec((R, 128), lambda i: (i, 0)),
            pl.BlockSpec((R, 128), lambda i: (i, 0)),
        ],
        out_shape=[
            jax.ShapeDtypeStruct((NPAD, 128), jnp.float32),
            jax.ShapeDtypeStruct((NPAD, 128), jnp.float32),
        ],
    )(s0, dinv, w1, sc1, sh1)


def _layer23_body(s1a_ref, s1b_ref, dinv_ref, w2_ref, w3_ref, sc2_ref,
                  sh2_ref, g2_ref):
    dv = dinv_ref[...]
    a2 = jnp.concatenate([s1a_ref[0] + s1a_ref[1],
                          s1b_ref[0] + s1b_ref[1]], axis=1) * dv
    h = jnp.dot(a2, w2_ref[...], preferred_element_type=jnp.float32)
    h = jnp.maximum(h * sc2_ref[...] + sh2_ref[...], 0.0)
    z = jnp.dot(h, w3_ref[...], preferred_element_type=jnp.float32)
    g2_ref[...] = z * dv


def _layer23(s1a, s1b, dinv, w2, w3p, sc2, sh2):
    return pl.pallas_call(
        _layer23_body,
        grid=(NPAD // R,),
        in_specs=[
            pl.BlockSpec((2, R, 128), lambda i: (0, i, 0)),
            pl.BlockSpec((2, R, 128), lambda i: (0, i, 0)),
            pl.BlockSpec((R, 1), lambda i: (i, 0)),
            pl.BlockSpec((D_H, D_H), lambda i: (0, 0)),
            pl.BlockSpec((D_H, 64), lambda i: (0, 0)),
            pl.BlockSpec((1, D_H), lambda i: (0, 0)),
            pl.BlockSpec((1, D_H), lambda i: (0, 0)),
        ],
        out_specs=pl.BlockSpec((R, 64), lambda i: (i, 0)),
        out_shape=jax.ShapeDtypeStruct((NPAD, 64), jnp.float32),
    )(s1a, s1b, dinv, w2, w3p, sc2, sh2)


def _final_body(s2_ref, dinv_ref, b3_ref, out_ref):
    a3 = (s2_ref[0] + s2_ref[1]) * dinv_ref[...] + b3_ref[...]
    m = jnp.max(a3, axis=1, keepdims=True)
    e = jnp.exp(a3 - m)
    lse = jnp.log(jnp.sum(e, axis=1, keepdims=True))
    out_ref[...] = (a3 - m - lse)[:, :D_OUT]


def _final(s2, dinv, b3p):
    return pl.pallas_call(
        _final_body,
        grid=(NPAD // R,),
        in_specs=[
            pl.BlockSpec((2, R, 64), lambda i: (0, i, 0)),
            pl.BlockSpec((R, 1), lambda i: (i, 0)),
            pl.BlockSpec((1, 64), lambda i: (0, 0)),
        ],
        out_specs=pl.BlockSpec((R, D_OUT), lambda i: (i, 0)),
        out_shape=jax.ShapeDtypeStruct((N, D_OUT), jnp.float32),
    )(s2, dinv, b3p)


def kernel(x, edge_index, W1, b1, W2, b2, W3, b3,
           bn1_w, bn1_b, bn2_w, bn2_b):
    f32 = jnp.float32
    x_pad = jnp.pad(x, ((0, NPAD - N), (0, 0)))

    src = edge_index[0].reshape(NW, EPW)
    dst = edge_index[1].reshape(NW, EPW)
    n_fill = EPW_PAD - EPW
    fill = (N + (jnp.arange(NW * n_fill, dtype=jnp.int32) % (NPAD - N))
            ).reshape(NW, n_fill)
    srcs = jnp.concatenate([src, fill], axis=1).reshape(NW, CHUNKS, K)
    dsts = jnp.concatenate([dst, fill], axis=1).reshape(NW, CHUNKS, K)
    idxs = jnp.stack([srcs, dsts], axis=2)

    c1 = bn1_w * (1.0 / jnp.sqrt(1.0 + BN_EPS))
    sc1 = c1.reshape(1, D_H)
    sh1 = (b1 * c1 + bn1_b).reshape(1, D_H)
    c2 = bn2_w * (1.0 / jnp.sqrt(1.0 + BN_EPS))
    sc2 = c2.reshape(1, D_H)
    sh2 = (b2 * c2 + bn2_b).reshape(1, D_H)
    w3p = jnp.pad(W3, ((0, 0), (0, 64 - D_OUT)))
    b3p = jnp.concatenate([b3, jnp.full((64 - D_OUT,), -1e30, f32)]
                          ).reshape(1, 64)

    zeros1 = jnp.zeros((NPAD,), f32)
    zeros128 = jnp.zeros((NPAD, 128), f32)
    zeros64 = jnp.zeros((NPAD, 64), f32)

    deg2 = _hist(idxs, zeros1)
    dinv, g0 = _prep(deg2[:, :, None], x_pad)
    s0 = _scatter128(g0, zeros128, idxs)
    g1a, g1b = _layer1(s0, dinv, W1, sc1, sh1)
    s1a, s1b = _scatter128x2(g1a, g1b, zeros128, idxs)
    g2 = _layer23(s1a, s1b, dinv, W2, w3p, sc2, sh2)
    s2 = _scatter64(g2, zeros64, idxs)
    return _final(s2, dinv, b3p)


# speedup vs baseline: 1.2171x; 1.0466x over previous
import functools

import jax
import jax.numpy as jnp
from jax import lax
from jax.experimental import pallas as pl
from jax.experimental.pallas import tpu as pltpu
from jax.experimental.pallas import tpu_sc as plsc

N = 10000
E = 320000
D_IN = 128
D_H = 256
D_OUT = 40
BN_EPS = 1e-5

NPAD = 10240
NW = 32
K = 128
EPW = E // NW
NBUF = 2
CHUNKS = 80
EPW_PAD = CHUNKS * K
ROWS_PER_TILE = NPAD // 16
R = 1024


def _hist_body(idxs, zeros1, out, ibuf, ones_v, acc, isem):
    c = lax.axis_index("c")
    s = lax.axis_index("s")
    w = s * 2 + c
    lo = s * ROWS_PER_TILE
    for i in range(K // 16):
        ones_v[pl.ds(i * 16, 16)] = jnp.ones((16,), jnp.float32)
    pltpu.sync_copy(zeros1.at[pl.ds(lo, ROWS_PER_TILE)],
                    acc.at[pl.ds(lo, ROWS_PER_TILE)])
    plsc.subcore_barrier()

    for p in range(4):
        pltpu.async_copy(idxs.at[w, p], ibuf.at[p], isem)

    @pl.loop(0, CHUNKS)
    def _(j):
        ib = lax.rem(j, 4)
        pltpu.make_async_copy(idxs.at[w, 0], ibuf.at[ib], isem).wait()
        pltpu.sync_copy(ones_v, acc.at[ibuf.at[ib, 1]], add=True)

        @pl.when(j + 4 < CHUNKS)
        def _():
            pltpu.async_copy(idxs.at[w, j + 4], ibuf.at[ib], isem)

    plsc.subcore_barrier()
    pltpu.sync_copy(acc.at[pl.ds(lo, ROWS_PER_TILE)],
                    out.at[c, pl.ds(lo, ROWS_PER_TILE)])


_hist = functools.partial(
    pl.kernel,
    out_type=jax.ShapeDtypeStruct((2, NPAD), jnp.float32),
    mesh=plsc.VectorSubcoreMesh(core_axis_name="c", subcore_axis_name="s"),
    scratch_types=[
        pltpu.VMEM((4, 2, K), jnp.int32),
        pltpu.VMEM((K,), jnp.float32),
        pltpu.VMEM_SHARED((NPAD,), jnp.float32),
        pltpu.SemaphoreType.DMA,
    ],
)(_hist_body)


def _one_pass(table, zeros, idxs, out, ibuf, rows, acc, gsem, isem, c, w, lo):
    @pl.when(c == 0)
    def _():
        pltpu.sync_copy(table.at[pl.ds(lo, ROWS_PER_TILE)],
                        acc.at[pl.ds(lo, ROWS_PER_TILE)])

    @pl.when(c != 0)
    def _():
        pltpu.sync_copy(zeros.at[pl.ds(lo, ROWS_PER_TILE)],
                        acc.at[pl.ds(lo, ROWS_PER_TILE)])

    plsc.subcore_barrier()

    for p in range(4):
        pltpu.async_copy(idxs.at[w, p], ibuf.at[p], isem)
    for p in range(NBUF):
        pltpu.make_async_copy(idxs.at[w, 0], ibuf.at[p], isem).wait()
        pltpu.async_copy(table.at[ibuf.at[p, 0]], rows.at[p], gsem)

    @pl.loop(0, CHUNKS, step=NBUF)
    def _(i):
        for b in range(NBUF):
            j = i + b
            ib = lax.rem(j, 4)
            pltpu.make_async_copy(table.at[pl.ds(0, K)], rows.at[b],
                                  gsem).wait()
            pltpu.sync_copy(rows.at[b], acc.at[ibuf.at[ib, 1]], add=True)

            @pl.when(j + 4 < CHUNKS)
            def _():
                pltpu.async_copy(idxs.at[w, j + 4], ibuf.at[ib], isem)

            @pl.when(j + NBUF < CHUNKS)
            def _():
                ib2 = lax.rem(j + NBUF, 4)
                pltpu.make_async_copy(idxs.at[w, 0], ibuf.at[ib2],
                                      isem).wait()
                pltpu.async_copy(table.at[ibuf.at[ib2, 0]], rows.at[b],
                                 gsem)

    plsc.subcore_barrier()
    pltpu.sync_copy(acc.at[pl.ds(lo, ROWS_PER_TILE)],
                    out.at[c, pl.ds(lo, ROWS_PER_TILE)])


def _make_scatter(D, n_tables=1, tc_tiling=True):
    def body(*args):
        tables = args[:n_tables]
        zeros, idxs = args[n_tables:n_tables + 2]
        outs = args[n_tables + 2:n_tables + 2 + n_tables]
        ibuf, rows, acc, gsem, isem = args[n_tables + 2 + n_tables:]
        c = lax.axis_index("c")
        s = lax.axis_index("s")
        w = s * 2 + c
        lo = s * ROWS_PER_TILE
        for t in range(n_tables):
            _one_pass(tables[t], zeros, idxs, outs[t], ibuf, rows, acc,
                      gsem, isem, c, w, lo)

    out_t = jax.ShapeDtypeStruct((2, NPAD, D), jnp.float32)
    return functools.partial(
        pl.kernel,
        out_type=out_t if n_tables == 1 else [out_t] * n_tables,
        mesh=plsc.VectorSubcoreMesh(core_axis_name="c", subcore_axis_name="s"),
        compiler_params=None if tc_tiling else pltpu.CompilerParams(
            use_tc_tiling_on_sc=False),
        scratch_types=[
            pltpu.VMEM((4, 2, K), jnp.int32),
            pltpu.VMEM((NBUF, K, D), jnp.float32),
            pltpu.VMEM_SHARED((NPAD, D), jnp.float32),
            pltpu.SemaphoreType.DMA,
            pltpu.SemaphoreType.DMA,
        ],
    )(body)


_scatter128 = _make_scatter(128)
_scatter128x2 = _make_scatter(128, n_tables=2)
_scatter64 = _make_scatter(64, tc_tiling=False)


def _prep_body(deg2_ref, x_ref, dinv_ref, g0_ref):
    deg = deg2_ref[0] + deg2_ref[1] + 1.0
    dv = lax.rsqrt(deg)
    dinv_ref[...] = dv
    g0_ref[...] = x_ref[...] * dv


def _prep(deg2, x_pad):
    return pl.pallas_call(
        _prep_body,
        grid=(NPAD // R,),
        in_specs=[
            pl.BlockSpec((2, R, 1), lambda i: (0, i, 0)),
            pl.BlockSpec((R, D_IN), lambda i: (i, 0)),
        ],
        out_specs=[
            pl.BlockSpec((R, 1), lambda i: (i, 0)),
            pl.BlockSpec((R, D_IN), lambda i: (i, 0)),
        ],
        out_shape=[
            jax.ShapeDtypeStruct((NPAD, 1), jnp.float32),
            jax.ShapeDtypeStruct((NPAD, D_IN), jnp.float32),
        ],
    )(deg2, x_pad)


def _layer1_body(s0_ref, dinv_ref, w1_ref, sc1_ref, sh1_ref, ga_ref, gb_ref):
    dv = dinv_ref[...]
    a1 = (s0_ref[0] + s0_ref[1]) * dv
    h = jnp.dot(a1, w1_ref[...], preferred_element_type=jnp.float32)
    h = jnp.maximum(h * sc1_ref[...] + sh1_ref[...], 0.0)
    g1 = h * dv
    ga_ref[...] = g1[:, :128]
    gb_ref[...] = g1[:, 128:]


def _layer1(s0, dinv, w1, sc1, sh1):
    return pl.pallas_call(
        _layer1_body,
        grid=(NPAD // R,),
        in_specs=[
            pl.BlockSpec((2, R, 128), lambda i: (0, i, 0)),
            pl.BlockSpec((R, 1), lambda i: (i, 0)),
            pl.BlockSpec((D_IN, D_H), lambda i: (0, 0)),
            pl.BlockSpec((1, D_H), lambda i: (0, 0)),
            pl.BlockSpec((1, D_H), lambda i: (0, 0)),
        ],
        out_specs=[
            pl.BlockSpec((R, 128), lambda i: (i, 0)),
            pl.BlockSpec((R, 128), lambda i: (i, 0)),
        ],
        out_shape=[
            jax.ShapeDtypeStruct((NPAD, 128), jnp.float32),
            jax.ShapeDtypeStruct((NPAD, 128), jnp.float32),
        ],
    )(s0, dinv, w1, sc1, sh1)


def _layer23_body(s1a_ref, s1b_ref, dinv_ref, w2_ref, w3_ref, sc2_ref,
                  sh2_ref, g2_ref):
    dv = dinv_ref[...]
    a2 = jnp.concatenate([s1a_ref[0] + s1a_ref[1],
                          s1b_ref[0] + s1b_ref[1]], axis=1) * dv
    h = jnp.dot(a2, w2_ref[...], preferred_element_type=jnp.float32)
    h = jnp.maximum(h * sc2_ref[...] + sh2_ref[...], 0.0)
    z = jnp.dot(h, w3_ref[...], preferred_element_type=jnp.float32)
    g2_ref[...] = z * dv


def _layer23(s1a, s1b, dinv, w2, w3p, sc2, sh2):
    return pl.pallas_call(
        _layer23_body,
        grid=(NPAD // R,),
        in_specs=[
            pl.BlockSpec((2, R, 128), lambda i: (0, i, 0)),
            pl.BlockSpec((2, R, 128), lambda i: (0, i, 0)),
            pl.BlockSpec((R, 1), lambda i: (i, 0)),
            pl.BlockSpec((D_H, D_H), lambda i: (0, 0)),
            pl.BlockSpec((D_H, 64), lambda i: (0, 0)),
            pl.BlockSpec((1, D_H), lambda i: (0, 0)),
            pl.BlockSpec((1, D_H), lambda i: (0, 0)),
        ],
        out_specs=pl.BlockSpec((R, 64), lambda i: (i, 0)),
        out_shape=jax.ShapeDtypeStruct((NPAD, 64), jnp.float32),
    )(s1a, s1b, dinv, w2, w3p, sc2, sh2)


def _final_body(s2_ref, dinv_ref, b3_ref, out_ref):
    a3 = (s2_ref[0] + s2_ref[1]) * dinv_ref[...] + b3_ref[...]
    m = jnp.max(a3, axis=1, keepdims=True)
    e = jnp.exp(a3 - m)
    lse = jnp.log(jnp.sum(e, axis=1, keepdims=True))
    out_ref[...] = (a3 - m - lse)[:, :D_OUT]


def _final(s2, dinv, b3p):
    return pl.pallas_call(
        _final_body,
        grid=(NPAD // R,),
        in_specs=[
            pl.BlockSpec((2, R, 64), lambda i: (0, i, 0)),
            pl.BlockSpec((R, 1), lambda i: (i, 0)),
            pl.BlockSpec((1, 64), lambda i: (0, 0)),
        ],
        out_specs=pl.BlockSpec((R, D_OUT), lambda i: (i, 0)),
        out_shape=jax.ShapeDtypeStruct((N, D_OUT), jnp.float32),
    )(s2, dinv, b3p)


def kernel(x, edge_index, W1, b1, W2, b2, W3, b3,
           bn1_w, bn1_b, bn2_w, bn2_b):
    f32 = jnp.float32
    x_pad = jnp.pad(x, ((0, NPAD - N), (0, 0)))

    src = edge_index[0].reshape(NW, EPW)
    dst = edge_index[1].reshape(NW, EPW)
    n_fill = EPW_PAD - EPW
    fill = (N + (jnp.arange(NW * n_fill, dtype=jnp.int32) % (NPAD - N))
            ).reshape(NW, n_fill)
    srcs = jnp.concatenate([src, fill], axis=1).reshape(NW, CHUNKS, K)
    dsts = jnp.concatenate([dst, fill], axis=1).reshape(NW, CHUNKS, K)
    idxs = jnp.stack([srcs, dsts], axis=2)

    c1 = bn1_w * (1.0 / jnp.sqrt(1.0 + BN_EPS))
    sc1 = c1.reshape(1, D_H)
    sh1 = (b1 * c1 + bn1_b).reshape(1, D_H)
    c2 = bn2_w * (1.0 / jnp.sqrt(1.0 + BN_EPS))
    sc2 = c2.reshape(1, D_H)
    sh2 = (b2 * c2 + bn2_b).reshape(1, D_H)
    w3p = jnp.pad(W3, ((0, 0), (0, 64 - D_OUT)))
    b3p = jnp.concatenate([b3, jnp.full((64 - D_OUT,), -1e30, f32)]
                          ).reshape(1, 64)

    zeros1 = jnp.zeros((NPAD,), f32)
    zeros128 = jnp.zeros((NPAD, 128), f32)
    zeros64 = jnp.zeros((NPAD, 64), f32)

    deg2 = _hist(idxs, zeros1)
    dinv, g0 = _prep(deg2[:, :, None], x_pad)
    s0 = _scatter128(g0, zeros128, idxs)
    g1a, g1b = _layer1(s0, dinv, W1, sc1, sh1)
    s1a, s1b = _scatter128x2(g1a, g1b, zeros128, idxs)
    g2 = _layer23(s1a, s1b, dinv, W2, w3p, sc2, sh2)
    s2 = _scatter64(g2, zeros64, idxs)
    return _final(s2, dinv, b3p)


# speedup vs baseline: 1.2328x; 1.0129x over previous
import functools

import jax
import jax.numpy as jnp
from jax import lax
from jax.experimental import pallas as pl
from jax.experimental.pallas import tpu as pltpu
from jax.experimental.pallas import tpu_sc as plsc

N = 10000
E = 320000
D_IN = 128
D_H = 256
D_OUT = 40
BN_EPS = 1e-5

NPAD = 10240
NW = 32
K = 128
EPW = E // NW
NBUF = 2
CHUNKS = 80
EPW_PAD = CHUNKS * K
ROWS_PER_TILE = NPAD // 16
R = 2048


def _hist_body(idxs, zeros1, out, ibuf, ones_v, acc, isem):
    c = lax.axis_index("c")
    s = lax.axis_index("s")
    w = s * 2 + c
    lo = s * ROWS_PER_TILE
    for i in range(K // 16):
        ones_v[pl.ds(i * 16, 16)] = jnp.ones((16,), jnp.float32)
    pltpu.sync_copy(zeros1.at[pl.ds(lo, ROWS_PER_TILE)],
                    acc.at[pl.ds(lo, ROWS_PER_TILE)])
    plsc.subcore_barrier()

    for p in range(4):
        pltpu.async_copy(idxs.at[w, p], ibuf.at[p], isem)

    @pl.loop(0, CHUNKS)
    def _(j):
        ib = lax.rem(j, 4)
        pltpu.make_async_copy(idxs.at[w, 0], ibuf.at[ib], isem).wait()
        pltpu.sync_copy(ones_v, acc.at[ibuf.at[ib, 1]], add=True)

        @pl.when(j + 4 < CHUNKS)
        def _():
            pltpu.async_copy(idxs.at[w, j + 4], ibuf.at[ib], isem)

    plsc.subcore_barrier()
    pltpu.sync_copy(acc.at[pl.ds(lo, ROWS_PER_TILE)],
                    out.at[c, pl.ds(lo, ROWS_PER_TILE)])


_hist = functools.partial(
    pl.kernel,
    out_type=jax.ShapeDtypeStruct((2, NPAD), jnp.float32),
    mesh=plsc.VectorSubcoreMesh(core_axis_name="c", subcore_axis_name="s"),
    scratch_types=[
        pltpu.VMEM((4, 2, K), jnp.int32),
        pltpu.VMEM((K,), jnp.float32),
        pltpu.VMEM_SHARED((NPAD,), jnp.float32),
        pltpu.SemaphoreType.DMA,
    ],
)(_hist_body)


def _one_pass(table, zeros, idxs, out, ibuf, rows, acc, gsem, isem, c, w, lo):
    @pl.when(c == 0)
    def _():
        pltpu.sync_copy(table.at[pl.ds(lo, ROWS_PER_TILE)],
                        acc.at[pl.ds(lo, ROWS_PER_TILE)])

    @pl.when(c != 0)
    def _():
        pltpu.sync_copy(zeros.at[pl.ds(lo, ROWS_PER_TILE)],
                        acc.at[pl.ds(lo, ROWS_PER_TILE)])

    plsc.subcore_barrier()

    for p in range(4):
        pltpu.async_copy(idxs.at[w, p], ibuf.at[p], isem)
    for p in range(NBUF):
        pltpu.make_async_copy(idxs.at[w, 0], ibuf.at[p], isem).wait()
        pltpu.async_copy(table.at[ibuf.at[p, 0]], rows.at[p], gsem)

    @pl.loop(0, CHUNKS, step=NBUF)
    def _(i):
        for b in range(NBUF):
            j = i + b
            ib = lax.rem(j, 4)
            pltpu.make_async_copy(table.at[pl.ds(0, K)], rows.at[b],
                                  gsem).wait()
            pltpu.sync_copy(rows.at[b], acc.at[ibuf.at[ib, 1]], add=True)

            @pl.when(j + 4 < CHUNKS)
            def _():
                pltpu.async_copy(idxs.at[w, j + 4], ibuf.at[ib], isem)

            @pl.when(j + NBUF < CHUNKS)
            def _():
                ib2 = lax.rem(j + NBUF, 4)
                pltpu.make_async_copy(idxs.at[w, 0], ibuf.at[ib2],
                                      isem).wait()
                pltpu.async_copy(table.at[ibuf.at[ib2, 0]], rows.at[b],
                                 gsem)

    plsc.subcore_barrier()
    pltpu.sync_copy(acc.at[pl.ds(lo, ROWS_PER_TILE)],
                    out.at[c, pl.ds(lo, ROWS_PER_TILE)])


def _make_scatter(D, n_tables=1, tc_tiling=True):
    def body(*args):
        tables = args[:n_tables]
        zeros, idxs = args[n_tables:n_tables + 2]
        outs = args[n_tables + 2:n_tables + 2 + n_tables]
        ibuf, rows, acc, gsem, isem = args[n_tables + 2 + n_tables:]
        c = lax.axis_index("c")
        s = lax.axis_index("s")
        w = s * 2 + c
        lo = s * ROWS_PER_TILE
        for t in range(n_tables):
            _one_pass(tables[t], zeros, idxs, outs[t], ibuf, rows, acc,
                      gsem, isem, c, w, lo)

    out_t = jax.ShapeDtypeStruct((2, NPAD, D), jnp.float32)
    return functools.partial(
        pl.kernel,
        out_type=out_t if n_tables == 1 else [out_t] * n_tables,
        mesh=plsc.VectorSubcoreMesh(core_axis_name="c", subcore_axis_name="s"),
        compiler_params=None if tc_tiling else pltpu.CompilerParams(
            use_tc_tiling_on_sc=False),
        scratch_types=[
            pltpu.VMEM((4, 2, K), jnp.int32),
            pltpu.VMEM((NBUF, K, D), jnp.float32),
            pltpu.VMEM_SHARED((NPAD, D), jnp.float32),
            pltpu.SemaphoreType.DMA,
            pltpu.SemaphoreType.DMA,
        ],
    )(body)


_scatter128 = _make_scatter(128)
_scatter128x2 = _make_scatter(128, n_tables=2)
_scatter64 = _make_scatter(64, tc_tiling=False)


def _prep_body(deg2_ref, x_ref, dinv_ref, g0_ref):
    deg = deg2_ref[0] + deg2_ref[1] + 1.0
    dv = lax.rsqrt(deg)
    dinv_ref[...] = dv
    g0_ref[...] = x_ref[...] * dv


def _prep(deg2, x_pad):
    return pl.pallas_call(
        _prep_body,
        grid=(NPAD // R,),
        in_specs=[
            pl.BlockSpec((2, R, 1), lambda i: (0, i, 0)),
            pl.BlockSpec((R, D_IN), lambda i: (i, 0)),
        ],
        out_specs=[
            pl.BlockSpec((R, 1), lambda i: (i, 0)),
            pl.BlockSpec((R, D_IN), lambda i: (i, 0)),
        ],
        out_shape=[
            jax.ShapeDtypeStruct((NPAD, 1), jnp.float32),
            jax.ShapeDtypeStruct((NPAD, D_IN), jnp.float32),
        ],
    )(deg2, x_pad)


def _layer1_body(s0_ref, dinv_ref, w1_ref, sc1_ref, sh1_ref, ga_ref, gb_ref):
    dv = dinv_ref[...]
    a1 = (s0_ref[0] + s0_ref[1]) * dv
    h = jnp.dot(a1, w1_ref[...], preferred_element_type=jnp.float32)
    h = jnp.maximum(h * sc1_ref[...] + sh1_ref[...], 0.0)
    g1 = h * dv
    ga_ref[...] = g1[:, :128]
    gb_ref[...] = g1[:, 128:]


def _layer1(s0, dinv, w1, sc1, sh1):
    return pl.pallas_call(
        _layer1_body,
        grid=(NPAD // R,),
        in_specs=[
            pl.BlockSpec((2, R, 128), lambda i: (0, i, 0)),
            pl.BlockSpec((R, 1), lambda i: (i, 0)),
            pl.BlockSpec((D_IN, D_H), lambda i: (0, 0)),
            pl.BlockSpec((1, D_H), lambda i: (0, 0)),
            pl.BlockSpec((1, D_H), lambda i: (0, 0)),
        ],
        out_specs=[
            pl.BlockSpec((R, 128), lambda i: (i, 0)),
            pl.BlockSpec((R, 128), lambda i: (i, 0)),
        ],
        out_shape=[
            jax.ShapeDtypeStruct((NPAD, 128), jnp.float32),
            jax.ShapeDtypeStruct((NPAD, 128), jnp.float32),
        ],
    )(s0, dinv, w1, sc1, sh1)


def _layer23_body(s1a_ref, s1b_ref, dinv_ref, w2_ref, w3_ref, sc2_ref,
                  sh2_ref, g2_ref):
    dv = dinv_ref[...]
    a2 = jnp.concatenate([s1a_ref[0] + s1a_ref[1],
                          s1b_ref[0] + s1b_ref[1]], axis=1) * dv
    h = jnp.dot(a2, w2_ref[...], preferred_element_type=jnp.float32)
    h = jnp.maximum(h * sc2_ref[...] + sh2_ref[...], 0.0)
    z = jnp.dot(h, w3_ref[...], preferred_element_type=jnp.float32)
    g2_ref[...] = z * dv


def _layer23(s1a, s1b, dinv, w2, w3p, sc2, sh2):
    return pl.pallas_call(
        _layer23_body,
        grid=(NPAD // R,),
        in_specs=[
            pl.BlockSpec((2, R, 128), lambda i: (0, i, 0)),
            pl.BlockSpec((2, R, 128), lambda i: (0, i, 0)),
            pl.BlockSpec((R, 1), lambda i: (i, 0)),
            pl.BlockSpec((D_H, D_H), lambda i: (0, 0)),
            pl.BlockSpec((D_H, 64), lambda i: (0, 0)),
            pl.BlockSpec((1, D_H), lambda i: (0, 0)),
            pl.BlockSpec((1, D_H), lambda i: (0, 0)),
        ],
        out_specs=pl.BlockSpec((R, 64), lambda i: (i, 0)),
        out_shape=jax.ShapeDtypeStruct((NPAD, 64), jnp.float32),
    )(s1a, s1b, dinv, w2, w3p, sc2, sh2)


def _final_body(s2_ref, dinv_ref, b3_ref, out_ref):
    a3 = (s2_ref[0] + s2_ref[1]) * dinv_ref[...] + b3_ref[...]
    m = jnp.max(a3, axis=1, keepdims=True)
    e = jnp.exp(a3 - m)
    lse = jnp.log(jnp.sum(e, axis=1, keepdims=True))
    out_ref[...] = (a3 - m - lse)[:, :D_OUT]


def _final(s2, dinv, b3p):
    return pl.pallas_call(
        _final_body,
        grid=(NPAD // R,),
        in_specs=[
            pl.BlockSpec((2, R, 64), lambda i: (0, i, 0)),
            pl.BlockSpec((R, 1), lambda i: (i, 0)),
            pl.BlockSpec((1, 64), lambda i: (0, 0)),
        ],
        out_specs=pl.BlockSpec((R, D_OUT), lambda i: (i, 0)),
        out_shape=jax.ShapeDtypeStruct((N, D_OUT), jnp.float32),
    )(s2, dinv, b3p)


def kernel(x, edge_index, W1, b1, W2, b2, W3, b3,
           bn1_w, bn1_b, bn2_w, bn2_b):
    f32 = jnp.float32
    x_pad = jnp.pad(x, ((0, NPAD - N), (0, 0)))

    src = edge_index[0].reshape(NW, EPW)
    dst = edge_index[1].reshape(NW, EPW)
    n_fill = EPW_PAD - EPW
    fill = (N + (jnp.arange(NW * n_fill, dtype=jnp.int32) % (NPAD - N))
            ).reshape(NW, n_fill)
    srcs = jnp.concatenate([src, fill], axis=1).reshape(NW, CHUNKS, K)
    dsts = jnp.concatenate([dst, fill], axis=1).reshape(NW, CHUNKS, K)
    idxs = jnp.stack([srcs, dsts], axis=2)

    c1 = bn1_w * (1.0 / jnp.sqrt(1.0 + BN_EPS))
    sc1 = c1.reshape(1, D_H)
    sh1 = (b1 * c1 + bn1_b).reshape(1, D_H)
    c2 = bn2_w * (1.0 / jnp.sqrt(1.0 + BN_EPS))
    sc2 = c2.reshape(1, D_H)
    sh2 = (b2 * c2 + bn2_b).reshape(1, D_H)
    w3p = jnp.pad(W3, ((0, 0), (0, 64 - D_OUT)))
    b3p = jnp.concatenate([b3, jnp.full((64 - D_OUT,), -1e30, f32)]
                          ).reshape(1, 64)

    zeros1 = jnp.zeros((NPAD,), f32)
    zeros128 = jnp.zeros((NPAD, 128), f32)
    zeros64 = jnp.zeros((NPAD, 64), f32)

    deg2 = _hist(idxs, zeros1)
    dinv, g0 = _prep(deg2[:, :, None], x_pad)
    s0 = _scatter128(g0, zeros128, idxs)
    g1a, g1b = _layer1(s0, dinv, W1, sc1, sh1)
    s1a, s1b = _scatter128x2(g1a, g1b, zeros128, idxs)
    g2 = _layer23(s1a, s1b, dinv, W2, w3p, sc2, sh2)
    s2 = _scatter64(g2, zeros64, idxs)
    return _final(s2, dinv, b3p)
